# Initial kernel scaffold; baseline (speedup 1.0000x reference)
import jax
import jax.numpy as jnp
from jax.experimental import pallas as pl


def kernel(node_attr, edge_attr, pair_indices, kernel, bias):
    raise NotImplementedError("write your pallas kernel here")


# speedup vs baseline: 2.2817x; 2.2817x over previous
import functools

import jax
import jax.numpy as jnp
from jax import lax
from jax.experimental import pallas as pl
from jax.experimental.pallas import tpu as pltpu
from jax.experimental.pallas import tpu_sc as plsc

NC = 2
NS = 16
NW = NC * NS


def _sc_mesh():
    return plsc.VectorSubcoreMesh(core_axis_name="c", subcore_axis_name="s")


def _make_gather(n_nodes, n_edges, d):
    epw = n_edges // NW
    c = min(epw, 1000)
    n_chunks = epw // c

    @functools.partial(
        pl.kernel,
        mesh=_sc_mesh(),
        out_type=jax.ShapeDtypeStruct((n_edges, d), jnp.float32),
        scratch_types=[
            pltpu.VMEM((c,), jnp.int32),
            pltpu.VMEM((c, d), jnp.float32),
            pltpu.SemaphoreType.DMA,
        ],
        compiler_params=pltpu.CompilerParams(use_tc_tiling_on_sc=False),
    )
    def gather_k(node_hbm, src_hbm, nb_hbm, idx_v, rows_v, sem):
        wid = lax.axis_index("s") * NC + lax.axis_index("c")
        for i in range(n_chunks):
            base = wid * epw + i * c
            pltpu.sync_copy(src_hbm.at[pl.ds(base, c)], idx_v)
            pltpu.async_copy(node_hbm.at[idx_v], rows_v, sem).wait()
            pltpu.sync_copy(rows_v, nb_hbm.at[pl.ds(base, c)])

    return gather_k


def _tc_transform_body(nb_ref, ea_ref, wt_ref, out_ref):
    y = jnp.dot(nb_ref[...], wt_ref[...], preferred_element_type=jnp.float32)
    d = nb_ref.shape[1]
    k = ea_ref.shape[1]
    acc = y[:, k * d:(k + 1) * d]
    for j in range(k):
        acc = acc + ea_ref[:, j:j + 1] * y[:, j * d:(j + 1) * d]
    out_ref[...] = acc


def _tc_transform(nb, edge_attr, wt):
    e, d = nb.shape
    be = 2000
    grid = e // be
    return pl.pallas_call(
        _tc_transform_body,
        grid=(grid,),
        in_specs=[
            pl.BlockSpec((be, d), lambda i: (i, 0)),
            pl.BlockSpec((be, edge_attr.shape[1]), lambda i: (i, 0)),
            pl.BlockSpec(wt.shape, lambda i: (0, 0)),
        ],
        out_specs=pl.BlockSpec((be, d), lambda i: (i, 0)),
        out_shape=jax.ShapeDtypeStruct((e, d), jnp.float32),
    )(nb, edge_attr, wt)


def _make_scatter(n_nodes, n_edges, d):
    epw = n_edges // NW
    c = min(epw, 1000)
    n_chunks = epw // c
    stripe = n_nodes // NS

    @functools.partial(
        pl.kernel,
        mesh=_sc_mesh(),
        out_type=jax.ShapeDtypeStruct((NC, n_nodes, d), jnp.float32),
        scratch_types=[
            pltpu.VMEM((c,), jnp.int32),
            pltpu.VMEM((c, d), jnp.float32),
            pltpu.VMEM((stripe, d), jnp.float32),
            pltpu.VMEM_SHARED((n_nodes, d), jnp.float32),
        ],
        compiler_params=pltpu.CompilerParams(use_tc_tiling_on_sc=False),
    )
    def scatter_k(rows_hbm, dst_hbm, out_hbm, idx_v, rows_v, zbuf_v, acc_sh):
        cid = lax.axis_index("c")
        sid = lax.axis_index("s")
        wid = sid * NC + cid

        zvec = jnp.zeros((16,), jnp.float32)

        def _zero_row(r, carry):
            for h in range(d // 16):
                zbuf_v[r, pl.ds(h * 16, 16)] = zvec
            return carry

        lax.fori_loop(0, stripe, _zero_row, 0)
        pltpu.sync_copy(zbuf_v, acc_sh.at[pl.ds(sid * stripe, stripe)])
        plsc.subcore_barrier()

        for i in range(n_chunks):
            base = wid * epw + i * c
            pltpu.sync_copy(dst_hbm.at[pl.ds(base, c)], idx_v)
            pltpu.sync_copy(rows_hbm.at[pl.ds(base, c)], rows_v)
            pltpu.sync_copy(rows_v, acc_sh.at[idx_v], add=True)

        plsc.subcore_barrier()
        pltpu.sync_copy(
            acc_sh.at[pl.ds(sid * stripe, stripe)],
            out_hbm.at[cid, pl.ds(sid * stripe, stripe)],
        )

    return scatter_k


def _tc_sum_body(a_ref, b_ref, out_ref):
    out_ref[...] = a_ref[...] + b_ref[...]


def _tc_sum(p0, p1):
    n, d = p0.shape
    bn = n // 5
    return pl.pallas_call(
        _tc_sum_body,
        grid=(n // bn,),
        in_specs=[
            pl.BlockSpec((bn, d), lambda i: (i, 0)),
            pl.BlockSpec((bn, d), lambda i: (i, 0)),
        ],
        out_specs=pl.BlockSpec((bn, d), lambda i: (i, 0)),
        out_shape=jax.ShapeDtypeStruct((n, d), jnp.float32),
    )(p0, p1)


def kernel(node_attr, edge_attr, pair_indices, kernel, bias):
    n_nodes, d = node_attr.shape
    n_edges, k = edge_attr.shape

    dst = pair_indices[:, 0].astype(jnp.int32)
    src = pair_indices[:, 1].astype(jnp.int32)

    wt_main = kernel.reshape(k, d, d).transpose(2, 0, 1).reshape(d, k * d)
    wt_bias = bias.reshape(d, d).T
    wt = jnp.concatenate([wt_main, wt_bias], axis=1)

    nb = _make_gather(n_nodes, n_edges, d)(node_attr, src)
    transformed = _tc_transform(nb, edge_attr, wt)
    parts = _make_scatter(n_nodes, n_edges, d)(transformed, dst)
    return _tc_sum(parts[0], parts[1])


# speedup vs baseline: 4.9926x; 2.1882x over previous
import functools

import jax
import jax.numpy as jnp
from jax import lax
from jax.experimental import pallas as pl
from jax.experimental.pallas import tpu as pltpu
from jax.experimental.pallas import tpu_sc as plsc

NC = 2
NS = 16
NW = NC * NS


def _sc_mesh():
    return plsc.VectorSubcoreMesh(core_axis_name="c", subcore_axis_name="s")


def _make_gather(n_nodes, n_edges, d):
    epw = n_edges // NW
    c = min(epw, 1000)
    n_chunks = epw // c

    @functools.partial(
        pl.kernel,
        mesh=_sc_mesh(),
        out_type=jax.ShapeDtypeStruct((n_edges, d), jnp.float32),
        scratch_types=[
            pltpu.VMEM((c,), jnp.int32),
            pltpu.VMEM((c, d), jnp.float32),
            pltpu.SemaphoreType.DMA,
        ],
        compiler_params=pltpu.CompilerParams(use_tc_tiling_on_sc=False),
    )
    def gather_k(node_hbm, src_hbm, nb_hbm, idx_v, rows_v, sem):
        wid = lax.axis_index("s") * NC + lax.axis_index("c")
        for i in range(n_chunks):
            base = wid * epw + i * c
            pltpu.sync_copy(src_hbm.at[pl.ds(base, c)], idx_v)
            pltpu.async_copy(node_hbm.at[idx_v], rows_v, sem).wait()
            pltpu.sync_copy(rows_v, nb_hbm.at[pl.ds(base, c)])

    return gather_k


def _tc_transform_body(nb_ref, eat_ref, wt_ref, g_ref, m1_ref, f_ref, out_ref):
    y = jnp.dot(nb_ref[...], wt_ref[...], preferred_element_type=jnp.float32)
    c = jax.lax.dot_general(
        eat_ref[...], g_ref[...],
        (((0,), (0,)), ((), ())),
        preferred_element_type=jnp.float32,
    ) + m1_ref[...]
    out_ref[...] = jnp.dot(y * c, f_ref[...], preferred_element_type=jnp.float32)


def _tc_transform(nb, eat, wt, g, m1, f):
    e, d = nb.shape
    be = 3200
    grid = e // be
    return pl.pallas_call(
        _tc_transform_body,
        grid=(grid,),
        in_specs=[
            pl.BlockSpec((be, d), lambda i: (i, 0)),
            pl.BlockSpec((eat.shape[0], be), lambda i: (0, i)),
            pl.BlockSpec(wt.shape, lambda i: (0, 0)),
            pl.BlockSpec(g.shape, lambda i: (0, 0)),
            pl.BlockSpec(m1.shape, lambda i: (0, 0)),
            pl.BlockSpec(f.shape, lambda i: (0, 0)),
        ],
        out_specs=pl.BlockSpec((be, d), lambda i: (i, 0)),
        out_shape=jax.ShapeDtypeStruct((e, d), jnp.float32),
    )(nb, eat, wt, g, m1, f)


def _make_scatter(n_nodes, n_edges, d):
    epw = n_edges // NW
    c = min(epw, 1000)
    n_chunks = epw // c
    stripe = n_nodes // NS

    @functools.partial(
        pl.kernel,
        mesh=_sc_mesh(),
        out_type=jax.ShapeDtypeStruct((NC, n_nodes, d), jnp.float32),
        scratch_types=[
            pltpu.VMEM((c,), jnp.int32),
            pltpu.VMEM((c, d), jnp.float32),
            pltpu.VMEM((stripe, d), jnp.float32),
            pltpu.VMEM_SHARED((n_nodes, d), jnp.float32),
        ],
        compiler_params=pltpu.CompilerParams(use_tc_tiling_on_sc=False),
    )
    def scatter_k(rows_hbm, dst_hbm, out_hbm, idx_v, rows_v, zbuf_v, acc_sh):
        cid = lax.axis_index("c")
        sid = lax.axis_index("s")
        wid = sid * NC + cid

        zvec = jnp.zeros((16,), jnp.float32)

        def _zero_row(r, carry):
            for h in range(d // 16):
                zbuf_v[r, pl.ds(h * 16, 16)] = zvec
            return carry

        lax.fori_loop(0, stripe, _zero_row, 0)
        pltpu.sync_copy(zbuf_v, acc_sh.at[pl.ds(sid * stripe, stripe)])
        plsc.subcore_barrier()

        for i in range(n_chunks):
            base = wid * epw + i * c
            pltpu.sync_copy(dst_hbm.at[pl.ds(base, c)], idx_v)
            pltpu.sync_copy(rows_hbm.at[pl.ds(base, c)], rows_v)
            pltpu.sync_copy(rows_v, acc_sh.at[idx_v], add=True)

        plsc.subcore_barrier()
        pltpu.sync_copy(
            acc_sh.at[pl.ds(sid * stripe, stripe)],
            out_hbm.at[cid, pl.ds(sid * stripe, stripe)],
        )

    return scatter_k


def _tc_sum_body(a_ref, b_ref, out_ref):
    out_ref[...] = a_ref[...] + b_ref[...]


def _tc_sum(p0, p1):
    n, d = p0.shape
    bn = n // 5
    return pl.pallas_call(
        _tc_sum_body,
        grid=(n // bn,),
        in_specs=[
            pl.BlockSpec((bn, d), lambda i: (i, 0)),
            pl.BlockSpec((bn, d), lambda i: (i, 0)),
        ],
        out_specs=pl.BlockSpec((bn, d), lambda i: (i, 0)),
        out_shape=jax.ShapeDtypeStruct((n, d), jnp.float32),
    )(p0, p1)


def kernel(node_attr, edge_attr, pair_indices, kernel, bias):
    n_nodes, d = node_attr.shape
    n_edges, k = edge_attr.shape

    dst = pair_indices[:, 0].astype(jnp.int32)
    src = pair_indices[:, 1].astype(jnp.int32)

    wt_main = kernel.reshape(k, d, d).transpose(2, 0, 1).reshape(d, k * d)
    wt_bias = bias.reshape(d, d).T
    wt = jnp.concatenate([wt_main, wt_bias], axis=1)

    kk = jnp.arange(k)
    cols = jnp.arange((k + 1) * d)
    g = (cols[None, :] // d == kk[:, None]).astype(jnp.float32)
    m1 = (cols[None, :] >= k * d).astype(jnp.float32)
    ii = jnp.arange(d)
    f = (cols[:, None] % d == ii[None, :]).astype(jnp.float32)

    eat = edge_attr.T
    nb = _make_gather(n_nodes, n_edges, d)(node_attr, src)
    transformed = _tc_transform(nb, eat, wt, g, m1, f)
    parts = _make_scatter(n_nodes, n_edges, d)(transformed, dst)
    return _tc_sum(parts[0], parts[1])


# speedup vs baseline: 5.3138x; 1.0643x over previous
import functools

import jax
import jax.numpy as jnp
from jax import lax
from jax.experimental import pallas as pl
from jax.experimental.pallas import tpu as pltpu
from jax.experimental.pallas import tpu_sc as plsc

NC = 2
NS = 16
NW = NC * NS


def _sc_mesh():
    return plsc.VectorSubcoreMesh(core_axis_name="c", subcore_axis_name="s")


def _make_gather(n_nodes, n_edges, d):
    epw = n_edges // NW
    c = min(epw, 1000)
    n_chunks = epw // c

    @functools.partial(
        pl.kernel,
        mesh=_sc_mesh(),
        out_type=jax.ShapeDtypeStruct((n_edges, d), jnp.float32),
        scratch_types=[
            pltpu.VMEM((c,), jnp.int32),
            pltpu.VMEM((c, d), jnp.float32),
            pltpu.SemaphoreType.DMA,
        ],
        compiler_params=pltpu.CompilerParams(use_tc_tiling_on_sc=False),
    )
    def gather_k(node_hbm, src_hbm, nb_hbm, idx_v, rows_v, sem):
        wid = lax.axis_index("s") * NC + lax.axis_index("c")
        for i in range(n_chunks):
            base = wid * epw + i * c
            pltpu.sync_copy(src_hbm.at[pl.ds(base, c)], idx_v)
            pltpu.async_copy(node_hbm.at[idx_v], rows_v, sem).wait()
            pltpu.sync_copy(rows_v, nb_hbm.at[pl.ds(base, c)])

    return gather_k


def _tc_transform_body(nbp_ref, eap_ref, wt_ref, g_ref, m1_ref, f_ref, out_ref):
    y = jnp.dot(nbp_ref[...], wt_ref[...], preferred_element_type=jnp.float32)
    c = jnp.dot(eap_ref[...], g_ref[...], preferred_element_type=jnp.float32)
    c = c + m1_ref[...]
    out_ref[...] = jnp.dot(y * c, f_ref[...], preferred_element_type=jnp.float32)


def _tc_transform(nbp, eap, wt, g, m1, f):
    ep, w = nbp.shape
    bp = 400
    grid = ep // bp
    return pl.pallas_call(
        _tc_transform_body,
        grid=(grid,),
        in_specs=[
            pl.BlockSpec((bp, w), lambda i: (i, 0)),
            pl.BlockSpec((bp, eap.shape[1]), lambda i: (i, 0)),
            pl.BlockSpec(wt.shape, lambda i: (0, 0)),
            pl.BlockSpec(g.shape, lambda i: (0, 0)),
            pl.BlockSpec(m1.shape, lambda i: (0, 0)),
            pl.BlockSpec(f.shape, lambda i: (0, 0)),
        ],
        out_specs=pl.BlockSpec((bp, w), lambda i: (i, 0)),
        out_shape=jax.ShapeDtypeStruct((ep, w), jnp.float32),
    )(nbp, eap, wt, g, m1, f)


def _make_scatter(n_nodes, n_edges, d):
    epw = n_edges // NW
    c = min(epw, 1000)
    n_chunks = epw // c
    stripe = n_nodes // NS

    @functools.partial(
        pl.kernel,
        mesh=_sc_mesh(),
        out_type=jax.ShapeDtypeStruct((NC, n_nodes, d), jnp.float32),
        scratch_types=[
            pltpu.VMEM((c,), jnp.int32),
            pltpu.VMEM((c, d), jnp.float32),
            pltpu.VMEM((stripe, d), jnp.float32),
            pltpu.VMEM_SHARED((n_nodes, d), jnp.float32),
        ],
        compiler_params=pltpu.CompilerParams(use_tc_tiling_on_sc=False),
    )
    def scatter_k(rows_hbm, dst_hbm, out_hbm, idx_v, rows_v, zbuf_v, acc_sh):
        cid = lax.axis_index("c")
        sid = lax.axis_index("s")
        wid = sid * NC + cid

        zvec = jnp.zeros((16,), jnp.float32)

        def _zero_row(r, carry):
            for h in range(d // 16):
                zbuf_v[r, pl.ds(h * 16, 16)] = zvec
            return carry

        lax.fori_loop(0, stripe, _zero_row, 0)
        pltpu.sync_copy(zbuf_v, acc_sh.at[pl.ds(sid * stripe, stripe)])
        plsc.subcore_barrier()

        for i in range(n_chunks):
            base = wid * epw + i * c
            pltpu.sync_copy(dst_hbm.at[pl.ds(base, c)], idx_v)
            pltpu.sync_copy(rows_hbm.at[pl.ds(base, c)], rows_v)
            pltpu.sync_copy(rows_v, acc_sh.at[idx_v], add=True)

        plsc.subcore_barrier()
        pltpu.sync_copy(
            acc_sh.at[pl.ds(sid * stripe, stripe)],
            out_hbm.at[cid, pl.ds(sid * stripe, stripe)],
        )

    return scatter_k


def _tc_sum_body(pp_ref, out_ref):
    out_ref[...] = pp_ref[0] + pp_ref[1]


def _tc_sum(pp):
    _, n, d = pp.shape
    return pl.pallas_call(
        _tc_sum_body,
        in_specs=[pl.BlockSpec((2, n, d), lambda: (0, 0, 0))],
        out_specs=pl.BlockSpec((n, d), lambda: (0, 0)),
        out_shape=jax.ShapeDtypeStruct((n, d), jnp.float32),
    )(pp)


def kernel(node_attr, edge_attr, pair_indices, kernel, bias):
    n_nodes, d = node_attr.shape
    n_edges, k = edge_attr.shape

    dst = pair_indices[:, 0].astype(jnp.int32)
    src = pair_indices[:, 1].astype(jnp.int32)

    wt_main = kernel.reshape(k, d, d).transpose(2, 0, 1).reshape(d, k * d)
    wt_bias = bias.reshape(d, d).T
    wt = jnp.concatenate([wt_main, wt_bias], axis=1)

    kk = jnp.arange(k)
    cols = jnp.arange((k + 1) * d)
    g = (cols[None, :] // d == kk[:, None]).astype(jnp.float32)
    ii = jnp.arange(d)
    f = (cols[:, None] % d == ii[None, :]).astype(jnp.float32)
    eye8 = jnp.eye(8, dtype=jnp.float32)
    wt_big = jnp.kron(eye8, wt)
    g_big = jnp.kron(eye8, g)
    f_big = jnp.kron(eye8, f)
    m1_big = jnp.tile((cols[None, :] >= k * d).astype(jnp.float32), (1, 8))

    eap = (edge_attr.T.reshape(k, n_edges // 8, 8)
           .transpose(1, 2, 0).reshape(n_edges // 8, 8 * k))
    nb = _make_gather(n_nodes, n_edges, d)(node_attr, src)
    nbp = nb.reshape(n_edges // 8, 8 * d)
    tp = _tc_transform(nbp, eap, wt_big, g_big, m1_big, f_big)
    transformed = tp.reshape(n_edges, d)
    parts = _make_scatter(n_nodes, n_edges, d)(transformed, dst)
    pp = parts.reshape(2, n_nodes // 4, 4 * d)
    outp = _tc_sum(pp)
    return outp.reshape(n_nodes, d)


# speedup vs baseline: 5.8460x; 1.1001x over previous
import functools

import jax
import jax.numpy as jnp
from jax import lax
from jax.experimental import pallas as pl
from jax.experimental.pallas import tpu as pltpu
from jax.experimental.pallas import tpu_sc as plsc

NC = 2
NS = 16
NW = NC * NS


def _sc_mesh():
    return plsc.VectorSubcoreMesh(core_axis_name="c", subcore_axis_name="s")


def _make_gather(n_nodes, n_edges, d, k):
    epw = n_edges // NW
    c = min(epw, 1000)
    n_chunks = epw // c

    @functools.partial(
        pl.kernel,
        mesh=_sc_mesh(),
        out_type=(
            jax.ShapeDtypeStruct((n_edges, d), jnp.float32),
            jax.ShapeDtypeStruct((n_edges, 2 * k), jnp.float32),
        ),
        scratch_types=[
            pltpu.VMEM((c,), jnp.int32),
            pltpu.VMEM((c, d), jnp.float32),
            pltpu.VMEM((k, c), jnp.float32),
            pltpu.VMEM((c, 2 * k), jnp.float32),
            pltpu.SemaphoreType.DMA,
        ],
        compiler_params=pltpu.CompilerParams(
            use_tc_tiling_on_sc=False, needs_layout_passes=False),
    )
    def gather_k(node_hbm, src_hbm, eat_hbm, nb_hbm, earm_hbm,
                 idx_v, rows_v, eat_v, earow_v, sem):
        wid = lax.axis_index("s") * NC + lax.axis_index("c")
        zvec = jnp.zeros((16,), jnp.float32)

        def _zero_row(r, carry):
            earow_v[r, pl.ds(0, k)] = zvec
            earow_v[r, pl.ds(k, k)] = zvec
            return carry

        lax.fori_loop(0, c, _zero_row, 0)
        row_iota = lax.iota(jnp.int32, 16)

        for i in range(n_chunks):
            base = wid * epw + i * c
            pltpu.sync_copy(src_hbm.at[pl.ds(base, c)], idx_v)
            pltpu.async_copy(node_hbm.at[idx_v], rows_v, sem).wait()
            pltpu.sync_copy(rows_v, nb_hbm.at[pl.ds(base, c)])
            pltpu.sync_copy(eat_hbm.at[:, pl.ds(base, c)], eat_v)

            def _tr8(g8, carry):
                e0 = g8 * 8
                for u in range(8):
                    e = e0 + u
                    col = jnp.full((16,), e, jnp.int32)
                    vals = plsc.load_gather(eat_v, [row_iota, col])
                    earow_v[e, pl.ds(0, k)] = vals
                return carry

            lax.fori_loop(0, c // 8, _tr8, 0)
            pltpu.sync_copy(earow_v, earm_hbm.at[pl.ds(base, c)])

    return gather_k


def _tc_transform_body(nbp_ref, eap_ref, wt_ref, g_ref, m1_ref, f_ref, out_ref):
    nbp = nbp_ref[...].astype(jnp.bfloat16)
    eap = eap_ref[...].astype(jnp.bfloat16)
    y = jnp.dot(nbp, wt_ref[...], preferred_element_type=jnp.float32)
    c = jnp.dot(eap, g_ref[...], preferred_element_type=jnp.float32)
    c = c + m1_ref[...]
    z = (y * c).astype(jnp.bfloat16)
    out_ref[...] = jnp.dot(z, f_ref[...], preferred_element_type=jnp.float32)


def _tc_transform(nbp, eap, wt, g, m1, f):
    ep, w = nbp.shape
    bp = 800
    grid = ep // bp
    return pl.pallas_call(
        _tc_transform_body,
        grid=(grid,),
        in_specs=[
            pl.BlockSpec((bp, w), lambda i: (i, 0)),
            pl.BlockSpec((bp, eap.shape[1]), lambda i: (i, 0)),
            pl.BlockSpec(wt.shape, lambda i: (0, 0)),
            pl.BlockSpec(g.shape, lambda i: (0, 0)),
            pl.BlockSpec(m1.shape, lambda i: (0, 0)),
            pl.BlockSpec(f.shape, lambda i: (0, 0)),
        ],
        out_specs=pl.BlockSpec((bp, w), lambda i: (i, 0)),
        out_shape=jax.ShapeDtypeStruct((ep, w), jnp.float32),
    )(nbp, eap, wt, g, m1, f)


def _make_scatter(n_nodes, n_edges, d):
    epw = n_edges // NW
    c = min(epw, 1000)
    n_chunks = epw // c
    stripe = n_nodes // NS

    @functools.partial(
        pl.kernel,
        mesh=_sc_mesh(),
        out_type=jax.ShapeDtypeStruct((NC, n_nodes, d), jnp.float32),
        scratch_types=[
            pltpu.VMEM((c,), jnp.int32),
            pltpu.VMEM((c, d), jnp.float32),
            pltpu.VMEM((stripe, d), jnp.float32),
            pltpu.VMEM_SHARED((n_nodes, d), jnp.float32),
        ],
        compiler_params=pltpu.CompilerParams(use_tc_tiling_on_sc=False),
    )
    def scatter_k(rows_hbm, dst_hbm, out_hbm, idx_v, rows_v, zbuf_v, acc_sh):
        cid = lax.axis_index("c")
        sid = lax.axis_index("s")
        wid = sid * NC + cid

        zvec = jnp.zeros((16,), jnp.float32)

        def _zero_row(r, carry):
            for h in range(d // 16):
                zbuf_v[r, pl.ds(h * 16, 16)] = zvec
            return carry

        lax.fori_loop(0, stripe, _zero_row, 0)
        pltpu.sync_copy(zbuf_v, acc_sh.at[pl.ds(sid * stripe, stripe)])
        plsc.subcore_barrier()

        for i in range(n_chunks):
            base = wid * epw + i * c
            pltpu.sync_copy(dst_hbm.at[pl.ds(base, c)], idx_v)
            pltpu.sync_copy(rows_hbm.at[pl.ds(base, c)], rows_v)
            pltpu.sync_copy(rows_v, acc_sh.at[idx_v], add=True)

        plsc.subcore_barrier()
        pltpu.sync_copy(
            acc_sh.at[pl.ds(sid * stripe, stripe)],
            out_hbm.at[cid, pl.ds(sid * stripe, stripe)],
        )

    return scatter_k


def _tc_sum_body(pp_ref, out_ref):
    out_ref[...] = pp_ref[0] + pp_ref[1]


def _tc_sum(pp):
    _, n, d = pp.shape
    return pl.pallas_call(
        _tc_sum_body,
        in_specs=[pl.BlockSpec((2, n, d), lambda: (0, 0, 0))],
        out_specs=pl.BlockSpec((n, d), lambda: (0, 0)),
        out_shape=jax.ShapeDtypeStruct((n, d), jnp.float32),
    )(pp)


def kernel(node_attr, edge_attr, pair_indices, kernel, bias):
    n_nodes, d = node_attr.shape
    n_edges, k = edge_attr.shape

    dst = pair_indices[:, 0].astype(jnp.int32)
    src = pair_indices[:, 1].astype(jnp.int32)

    wt_main = kernel.reshape(k, d, d).transpose(2, 0, 1).reshape(d, k * d)
    wt_bias = bias.reshape(d, d).T
    wt = jnp.concatenate([wt_main, wt_bias], axis=1)

    kk = jnp.arange(k)
    cols = jnp.arange((k + 1) * d)
    g = (cols[None, :] // d == kk[:, None]).astype(jnp.float32)
    ii = jnp.arange(d)
    f = (cols[:, None] % d == ii[None, :]).astype(jnp.float32)
    eye4 = jnp.eye(4, dtype=jnp.float32)
    wt_big = jnp.kron(eye4, wt).astype(jnp.bfloat16)
    gpad = jnp.concatenate([g, jnp.zeros((d - k, (k + 1) * d), jnp.float32)], 0)
    g_big = jnp.kron(eye4, gpad).astype(jnp.bfloat16)
    f_big = jnp.kron(eye4, f).astype(jnp.bfloat16)
    m1_big = jnp.tile((cols[None, :] >= k * d).astype(jnp.float32), (1, 4))

    nb, earm = _make_gather(n_nodes, n_edges, d, k)(node_attr, src, edge_attr.T)
    nbp = nb.reshape(n_edges // 4, 4 * d)
    eap = earm.reshape(n_edges // 4, 8 * k)
    tp = _tc_transform(nbp, eap, wt_big, g_big, m1_big, f_big)
    transformed = tp.reshape(n_edges, d)
    parts = _make_scatter(n_nodes, n_edges, d)(transformed, dst)
    pp = parts.reshape(2, n_nodes // 4, 4 * d)
    outp = _tc_sum(pp)
    return outp.reshape(n_nodes, d)


# speedup vs baseline: 5.9803x; 1.0230x over previous
import functools

import jax
import jax.numpy as jnp
from jax import lax
from jax.experimental import pallas as pl
from jax.experimental.pallas import tpu as pltpu
from jax.experimental.pallas import tpu_sc as plsc

NC = 2
NS = 16
NW = NC * NS


def _sc_mesh():
    return plsc.VectorSubcoreMesh(core_axis_name="c", subcore_axis_name="s")


def _make_gather(n_nodes, n_edges, d, k):
    epw = n_edges // NW
    c = min(epw, 1000)
    n_chunks = epw // c

    @functools.partial(
        pl.kernel,
        mesh=_sc_mesh(),
        out_type=(
            jax.ShapeDtypeStruct((n_edges, d), jnp.float32),
            jax.ShapeDtypeStruct((n_edges, 2 * k), jnp.float32),
        ),
        scratch_types=[
            pltpu.VMEM((c,), jnp.int32),
            pltpu.VMEM((c, d), jnp.float32),
            pltpu.VMEM((k, c), jnp.float32),
            pltpu.VMEM((c, 2 * k), jnp.float32),
            pltpu.SemaphoreType.DMA,
        ],
        compiler_params=pltpu.CompilerParams(
            use_tc_tiling_on_sc=False, needs_layout_passes=False),
    )
    def gather_k(node_hbm, src_hbm, eat_hbm, nb_hbm, earm_hbm,
                 idx_v, rows_v, eat_v, earow_v, sem):
        wid = lax.axis_index("s") * NC + lax.axis_index("c")
        row_iota = lax.iota(jnp.int32, 16)
        zvec = jnp.zeros((16,), jnp.float32)
        onehot = (row_iota == 0).astype(jnp.float32)

        def _zero_row(r, carry):
            earow_v[r, pl.ds(0, k)] = zvec
            earow_v[r, pl.ds(k, k)] = onehot
            return carry

        lax.fori_loop(0, c, _zero_row, 0)

        for i in range(n_chunks):
            base = wid * epw + i * c
            pltpu.sync_copy(src_hbm.at[pl.ds(base, c)], idx_v)
            cp = pltpu.async_copy(node_hbm.at[idx_v], rows_v, sem)
            pltpu.sync_copy(eat_hbm.at[:, pl.ds(base, c)], eat_v)

            def _tr(g16, col0):
                col = col0
                for u in range(16):
                    e = g16 * 16 + u
                    vals = plsc.load_gather(eat_v, [row_iota, col])
                    earow_v[e, pl.ds(0, k)] = vals
                    col = col + 1
                return col

            lax.fori_loop(0, c // 16, _tr, jnp.zeros((16,), jnp.int32))
            pltpu.sync_copy(earow_v, earm_hbm.at[pl.ds(base, c)])
            cp.wait()
            pltpu.sync_copy(rows_v, nb_hbm.at[pl.ds(base, c)])

    return gather_k


def _tc_transform_body(nbp_ref, eap_ref, wt_ref, g_ref, f_ref, out_ref):
    nbp = nbp_ref[...].astype(jnp.bfloat16)
    eap = eap_ref[...].astype(jnp.bfloat16)
    y = jnp.dot(nbp, wt_ref[...],
                preferred_element_type=jnp.float32).astype(jnp.bfloat16)
    c = jnp.dot(eap, g_ref[...],
                preferred_element_type=jnp.float32).astype(jnp.bfloat16)
    out_ref[...] = jnp.dot(y * c, f_ref[...], preferred_element_type=jnp.float32)


def _tc_transform(nbp, eap, wt, g, f):
    ep, w = nbp.shape
    bp = 800
    grid = ep // bp
    return pl.pallas_call(
        _tc_transform_body,
        grid=(grid,),
        in_specs=[
            pl.BlockSpec((bp, w), lambda i: (i, 0)),
            pl.BlockSpec((bp, eap.shape[1]), lambda i: (i, 0)),
            pl.BlockSpec(wt.shape, lambda i: (0, 0)),
            pl.BlockSpec(g.shape, lambda i: (0, 0)),
            pl.BlockSpec(f.shape, lambda i: (0, 0)),
        ],
        out_specs=pl.BlockSpec((bp, w), lambda i: (i, 0)),
        out_shape=jax.ShapeDtypeStruct((ep, w), jnp.float32),
    )(nbp, eap, wt, g, f)


def _make_scatter(n_nodes, n_edges, d):
    epw = n_edges // NW
    c = min(epw, 1000)
    n_chunks = epw // c
    stripe = n_nodes // NS

    @functools.partial(
        pl.kernel,
        mesh=_sc_mesh(),
        out_type=jax.ShapeDtypeStruct((NC, n_nodes, d), jnp.float32),
        scratch_types=[
            pltpu.VMEM((c,), jnp.int32),
            pltpu.VMEM((c, d), jnp.float32),
            pltpu.VMEM((stripe, d), jnp.float32),
            pltpu.VMEM_SHARED((n_nodes, d), jnp.float32),
        ],
        compiler_params=pltpu.CompilerParams(use_tc_tiling_on_sc=False),
    )
    def scatter_k(rows_hbm, dst_hbm, out_hbm, idx_v, rows_v, zbuf_v, acc_sh):
        cid = lax.axis_index("c")
        sid = lax.axis_index("s")
        wid = sid * NC + cid

        zvec = jnp.zeros((16,), jnp.float32)

        def _zero_row(r, carry):
            for h in range(d // 16):
                zbuf_v[r, pl.ds(h * 16, 16)] = zvec
            return carry

        lax.fori_loop(0, stripe, _zero_row, 0)
        pltpu.sync_copy(zbuf_v, acc_sh.at[pl.ds(sid * stripe, stripe)])
        plsc.subcore_barrier()

        for i in range(n_chunks):
            base = wid * epw + i * c
            pltpu.sync_copy(dst_hbm.at[pl.ds(base, c)], idx_v)
            pltpu.sync_copy(rows_hbm.at[pl.ds(base, c)], rows_v)
            pltpu.sync_copy(rows_v, acc_sh.at[idx_v], add=True)

        plsc.subcore_barrier()
        pltpu.sync_copy(
            acc_sh.at[pl.ds(sid * stripe, stripe)],
            out_hbm.at[cid, pl.ds(sid * stripe, stripe)],
        )

    return scatter_k


def _tc_sum_body(pp_ref, out_ref):
    out_ref[...] = pp_ref[0] + pp_ref[1]


def _tc_sum(pp):
    _, n, d = pp.shape
    return pl.pallas_call(
        _tc_sum_body,
        in_specs=[pl.BlockSpec((2, n, d), lambda: (0, 0, 0))],
        out_specs=pl.BlockSpec((n, d), lambda: (0, 0)),
        out_shape=jax.ShapeDtypeStruct((n, d), jnp.float32),
    )(pp)


def kernel(node_attr, edge_attr, pair_indices, kernel, bias):
    n_nodes, d = node_attr.shape
    n_edges, k = edge_attr.shape

    dst = pair_indices[:, 0].astype(jnp.int32)
    src = pair_indices[:, 1].astype(jnp.int32)

    wt_main = kernel.reshape(k, d, d).transpose(2, 0, 1).reshape(d, k * d)
    wt_bias = bias.reshape(d, d).T
    wt = jnp.concatenate([wt_main, wt_bias], axis=1)

    kk = jnp.arange(k)
    cols = jnp.arange((k + 1) * d)
    g = (cols[None, :] // d == kk[:, None]).astype(jnp.float32)
    ii = jnp.arange(d)
    f = (cols[:, None] % d == ii[None, :]).astype(jnp.float32)
    eye4 = jnp.eye(4, dtype=jnp.float32)
    wt_big = jnp.kron(eye4, wt).astype(jnp.bfloat16)
    m1row = (cols[None, :] >= k * d).astype(jnp.float32)
    gpad = jnp.concatenate(
        [g, m1row, jnp.zeros((d - k - 1, (k + 1) * d), jnp.float32)], 0)
    g_big = jnp.kron(eye4, gpad).astype(jnp.bfloat16)
    f_big = jnp.kron(eye4, f).astype(jnp.bfloat16)

    nb, earm = _make_gather(n_nodes, n_edges, d, k)(node_attr, src, edge_attr.T)
    nbp = nb.reshape(n_edges // 4, 4 * d)
    eap = earm.reshape(n_edges // 4, 8 * k)
    tp = _tc_transform(nbp, eap, wt_big, g_big, f_big)
    transformed = tp.reshape(n_edges, d)
    parts = _make_scatter(n_nodes, n_edges, d)(transformed, dst)
    pp = parts.reshape(2, n_nodes // 4, 4 * d)
    outp = _tc_sum(pp)
    return outp.reshape(n_nodes, d)
